# baseline (speedup 1.0000x reference)
import jax
import jax.numpy as jnp
from jax.experimental import pallas as pl
from jax.experimental.pallas import tpu as pltpu

_B, _D, _L = 16384, 64, 8
_COLS = 2048
_OCC_LEVELS = (0, 2, 3, 5, 6, 7)


def _cell(lh, lc, rh, rc, Ult_ref, Urt_ref, b_ref):
    g = (jnp.dot(Ult_ref[...], lh, preferred_element_type=jnp.float32)
         + jnp.dot(Urt_ref[...], rh, preferred_element_type=jnp.float32)
         + b_ref[...])
    t = jnp.tanh(g)
    si = 0.5 + 0.5 * t[:_D]
    sfl = 0.5 + 0.5 * t[_D : 2 * _D]
    sfr = 0.5 + 0.5 * t[2 * _D : 3 * _D]
    so = 0.5 + 0.5 * t[3 * _D : 4 * _D]
    tu = t[4 * _D :]
    c = si * tu + sfl * lc + sfr * rc
    h = so * jnp.tanh(c)
    return h, c


def _body(hn_ref, cn_ref,
          h0_ref, h2_ref, h3_ref, h5_ref, h6_ref, h7_ref,
          c0_ref, c2_ref, c3_ref, c5_ref, c6_ref, c7_ref,
          mUlt_ref, mUrt_ref, mb_ref, sUlt_ref, sUrt_ref, sb_ref,
          oh_ref, oc_ref):
    h, c = _cell(h0_ref[0], c0_ref[0], hn_ref[...], cn_ref[...],
                 mUlt_ref, mUrt_ref, mb_ref)
    for hl_ref, cl_ref in ((h2_ref, c2_ref), (h3_ref, c3_ref),
                           (h5_ref, c5_ref), (h6_ref, c6_ref),
                           (h7_ref, c7_ref)):
        h, c = _cell(h, c, hl_ref[0], cl_ref[0], sUlt_ref, sUrt_ref, sb_ref)
    oh_ref[...] = h
    oc_ref[...] = c


def _level_spec(p):
    return pl.BlockSpec((1, _D, _COLS), lambda i, _p=p: (_p, 0, i))


def _prep_weights(Ul, Ur, b):
    scale = jnp.concatenate([jnp.full((4 * _D,), 0.5, jnp.float32),
                             jnp.ones((_D,), jnp.float32)])
    Ult = (Ul * scale[None, :]).T
    Urt = (Ur * scale[None, :]).T
    bt = (b * scale).reshape(-1, 1)
    return Ult, Urt, bt


def kernel(h_new, c_new, h_levels, c_levels, merge_Ul, merge_Ur, merge_b,
           sum_Ul, sum_Ur, sum_b):
    hnT = h_new.T
    cnT = c_new.T
    hlT = jnp.transpose(h_levels, (0, 2, 1))
    clT = jnp.transpose(c_levels, (0, 2, 1))

    mUlt, mUrt, mb = _prep_weights(merge_Ul, merge_Ur, merge_b)
    sUlt, sUrt, sb = _prep_weights(sum_Ul, sum_Ur, sum_b)

    nb = _B // _COLS

    hT, cT = pl.pallas_call(
        _body,
        grid=(nb,),
        in_specs=(
            [pl.BlockSpec((_D, _COLS), lambda i: (0, i))] * 2
            + [_level_spec(p) for p in _OCC_LEVELS]
            + [_level_spec(p) for p in _OCC_LEVELS]
            + [
                pl.BlockSpec((5 * _D, _D), lambda i: (0, 0)),
                pl.BlockSpec((5 * _D, _D), lambda i: (0, 0)),
                pl.BlockSpec((5 * _D, 1), lambda i: (0, 0)),
                pl.BlockSpec((5 * _D, _D), lambda i: (0, 0)),
                pl.BlockSpec((5 * _D, _D), lambda i: (0, 0)),
                pl.BlockSpec((5 * _D, 1), lambda i: (0, 0)),
            ]
        ),
        out_specs=[
            pl.BlockSpec((_D, _COLS), lambda i: (0, i)),
            pl.BlockSpec((_D, _COLS), lambda i: (0, i)),
        ],
        out_shape=[
            jax.ShapeDtypeStruct((_D, _B), jnp.float32),
            jax.ShapeDtypeStruct((_D, _B), jnp.float32),
        ],
        compiler_params=pltpu.CompilerParams(
            dimension_semantics=("arbitrary",),
        ),
    )(hnT, cnT,
      *([hlT] * 6), *([clT] * 6),
      mUlt, mUrt, mb, sUlt, sUrt, sb)
    return (hT.T, cT.T)


# speedup vs baseline: 1.2268x; 1.2268x over previous
import jax
import jax.numpy as jnp
from jax.experimental import pallas as pl
from jax.experimental.pallas import tpu as pltpu

_B, _D, _L = 16384, 64, 8
_COLS = 2048
_OCC_LEVELS = (0, 2, 3, 5, 6, 7)


def _cell(lh, lc, rh, rc, Ut_ref, b_ref):
    x = jnp.concatenate([lh, rh], axis=0)
    g = (jnp.dot(Ut_ref[...], x, preferred_element_type=jnp.float32)
         + b_ref[...])
    t = jnp.tanh(g)
    si = 0.5 + 0.5 * t[:_D]
    sfl = 0.5 + 0.5 * t[_D : 2 * _D]
    sfr = 0.5 + 0.5 * t[2 * _D : 3 * _D]
    so = 0.5 + 0.5 * t[3 * _D : 4 * _D]
    tu = t[4 * _D :]
    c = si * tu + sfl * lc + sfr * rc
    h = so * jnp.tanh(c)
    return h, c


def _body(hn_ref, cn_ref,
          h0_ref, h2_ref, h3_ref, h5_ref, h6_ref, h7_ref,
          c0_ref, c2_ref, c3_ref, c5_ref, c6_ref, c7_ref,
          mUt_ref, mb_ref, sUt_ref, sb_ref,
          oh_ref, oc_ref):
    h, c = _cell(h0_ref[0], c0_ref[0], hn_ref[...], cn_ref[...],
                 mUt_ref, mb_ref)
    for hl_ref, cl_ref in ((h2_ref, c2_ref), (h3_ref, c3_ref),
                           (h5_ref, c5_ref), (h6_ref, c6_ref),
                           (h7_ref, c7_ref)):
        h, c = _cell(h, c, hl_ref[0], cl_ref[0], sUt_ref, sb_ref)
    oh_ref[...] = h
    oc_ref[...] = c


def _level_spec(p):
    return pl.BlockSpec((1, _D, _COLS), lambda i, _p=p: (_p, 0, i))


def _prep_weights(Ul, Ur, b):
    scale = jnp.concatenate([jnp.full((4 * _D,), 0.5, jnp.float32),
                             jnp.ones((_D,), jnp.float32)])
    Ut = (jnp.concatenate([Ul, Ur], axis=0) * scale[None, :]).T
    bt = (b * scale).reshape(-1, 1)
    return Ut, bt


def kernel(h_new, c_new, h_levels, c_levels, merge_Ul, merge_Ur, merge_b,
           sum_Ul, sum_Ur, sum_b):
    hnT = h_new.T
    cnT = c_new.T
    hlT = jnp.transpose(h_levels, (0, 2, 1))
    clT = jnp.transpose(c_levels, (0, 2, 1))

    mUt, mb = _prep_weights(merge_Ul, merge_Ur, merge_b)
    sUt, sb = _prep_weights(sum_Ul, sum_Ur, sum_b)

    nb = _B // _COLS

    hT, cT = pl.pallas_call(
        _body,
        grid=(nb,),
        in_specs=(
            [pl.BlockSpec((_D, _COLS), lambda i: (0, i))] * 2
            + [_level_spec(p) for p in _OCC_LEVELS]
            + [_level_spec(p) for p in _OCC_LEVELS]
            + [
                pl.BlockSpec((5 * _D, 2 * _D), lambda i: (0, 0)),
                pl.BlockSpec((5 * _D, 1), lambda i: (0, 0)),
                pl.BlockSpec((5 * _D, 2 * _D), lambda i: (0, 0)),
                pl.BlockSpec((5 * _D, 1), lambda i: (0, 0)),
            ]
        ),
        out_specs=[
            pl.BlockSpec((_D, _COLS), lambda i: (0, i)),
            pl.BlockSpec((_D, _COLS), lambda i: (0, i)),
        ],
        out_shape=[
            jax.ShapeDtypeStruct((_D, _B), jnp.float32),
            jax.ShapeDtypeStruct((_D, _B), jnp.float32),
        ],
        compiler_params=pltpu.CompilerParams(
            dimension_semantics=("arbitrary",),
        ),
    )(hnT, cnT,
      *([hlT] * 6), *([clT] * 6),
      mUt, mb, sUt, sb)
    return (hT.T, cT.T)
